# baseline (speedup 1.0000x reference)
import functools

import jax
import jax.numpy as jnp
from jax import lax
from jax.experimental import pallas as pl
from jax.experimental.pallas import tpu as pltpu
from jax.experimental.pallas import tpu_sc as plsc

BS, C, H, W = 8, 9, 512, 512
NPIX = BS * H * W

HB = 64
NHB = H // HB

MANT_BITS = 6
EXP_LO = 102
EXP_HI = 131
NBUCK = (EXP_HI - EXP_LO + 1) * (1 << MANT_BITS)
BUCK_BASE = EXP_LO << MANT_BITS

NW = 32
LANES = 16
NSLOT = NBUCK * LANES
QROWS = H // 4
CROWS = 8
CPIX = CROWS * W
NCHUNK = QROWS // CROWS


def _sl1(p, t):
    d = jnp.abs(p - t)
    return jnp.where(d < 1.0, 0.5 * d * d, d - 0.5)


def _main_body(c4_ref, c5_ref, c6_ref, c7_ref, up_ref, dn_ref, lf_ref, rt_ref,
               acc_ref):
    sums = [
        jnp.sum(_sl1(c4_ref[0, 0], up_ref[0])),
        jnp.sum(_sl1(c5_ref[0, 0], dn_ref[0])),
        jnp.sum(_sl1(c6_ref[0, 0], lf_ref[0])),
        jnp.sum(_sl1(c7_ref[0, 0], rt_ref[0])),
    ]
    rows = lax.broadcasted_iota(jnp.int32, (8, 128), 0)
    cols = lax.broadcasted_iota(jnp.int32, (8, 128), 1)
    p = jnp.zeros((8, 128), jnp.float32)
    for i, s in enumerate(sums):
        p += jnp.where((rows == i) & (cols == 0), s, 0.0)

    @pl.when((pl.program_id(0) == 0) & (pl.program_id(1) == 0))
    def _():
        acc_ref[...] = jnp.zeros_like(acc_ref)

    acc_ref[...] += p


def _stage1(inp, up, dn, lf, rt):
    def chan(c):
        return pl.BlockSpec((1, 1, HB, W), lambda b, h, c=c: (b, c, h, 0))

    msk = pl.BlockSpec((1, HB, W), lambda b, h: (b, h, 0))
    return pl.pallas_call(
        _main_body,
        grid=(BS, NHB),
        in_specs=[chan(4), chan(5), chan(6), chan(7)] + [msk] * 4,
        out_specs=pl.BlockSpec((8, 128), lambda b, h: (0, 0)),
        out_shape=jax.ShapeDtypeStruct((8, 128), jnp.float32),
    )(inp, inp, inp, inp, up, dn, lf, rt)


def _softplus_sc(m):
    e = jnp.exp(-jnp.abs(m))
    p = jnp.float32(-0.017414081461728154)
    for c in (0.08269125547625317, -0.19035436664917366, 0.3157473388836618,
              -0.49737322382928023, 0.9998476986074015, 1.4720198885558882e-06):
        p = p * e + jnp.float32(c)
    return jnp.maximum(m, 0.0) + p


def _sc_main(inp, tr, tcl, w, trm):
    mesh = plsc.VectorSubcoreMesh(core_axis_name="c", subcore_axis_name="s")
    fbuf = lambda: pltpu.VMEM((CROWS, W), jnp.float32)
    ibuf = lambda: pltpu.VMEM((CROWS, W), jnp.int32)

    @functools.partial(
        pl.kernel,
        out_type=(jax.ShapeDtypeStruct((NW, NSLOT), jnp.float32),
                  jax.ShapeDtypeStruct((NW, NSLOT), jnp.float32),
                  jax.ShapeDtypeStruct((NW, 6, 16), jnp.float32)),
        mesh=mesh,
        scratch_types=[
            fbuf(), fbuf(), fbuf(), fbuf(),
            fbuf(), fbuf(), fbuf(), fbuf(),
            ibuf(), ibuf(), fbuf(), ibuf(),
            ibuf(), ibuf(), fbuf(), ibuf(),
            pltpu.VMEM((NSLOT,), jnp.float32),
            pltpu.VMEM((NSLOT,), jnp.float32),
            pltpu.VMEM((6, 16), jnp.float32),
            pltpu.SemaphoreType.DMA,
            pltpu.SemaphoreType.DMA,
        ],
        compiler_params=pltpu.CompilerParams(needs_layout_passes=False),
    )
    def sc_kernel(in_hbm, tr_hbm, tcl_hbm, w_hbm, trm_hbm,
                  cnt_hbm, sum_hbm, scal_hbm,
                  l0a, l1a, l2a, l3a, l0b, l1b, l2b, l3b,
                  tra, tca, wa, trma, trb, tcb, wb, trmb,
                  cnt_v, sum_v, scal_v, sema, semb):
        wid = lax.axis_index("s") * 2 + lax.axis_index("c")
        b = wid // 4
        row0 = (wid % 4) * QROWS
        zeros16 = jnp.zeros((16,), jnp.float32)

        @plsc.parallel_loop(0, NSLOT // 16, 1, unroll=8)
        def _zero(i):
            cnt_v[pl.ds(i * 16, 16)] = zeros16
            sum_v[pl.ds(i * 16, 16)] = zeros16

        lane = lax.iota(jnp.int32, 16)
        ones16 = jnp.ones((16,), jnp.float32)
        slots = ((l0a, l1a, l2a, l3a, tra, tca, wa, trma, sema),
                 (l0b, l1b, l2b, l3b, trb, tcb, wb, trmb, semb))

        def _issue(slot, k):
            rows = pl.ds(row0 + k * CROWS, CROWS)
            l0, l1, l2, l3, trv, tcv, wv, trmv, sem = slots[slot]
            return (pltpu.async_copy(in_hbm.at[b, 0, rows, :], l0, sem),
                    pltpu.async_copy(in_hbm.at[b, 1, rows, :], l1, sem),
                    pltpu.async_copy(in_hbm.at[b, 2, rows, :], l2, sem),
                    pltpu.async_copy(in_hbm.at[b, 3, rows, :], l3, sem),
                    pltpu.async_copy(tr_hbm.at[b, rows, :], trv, sem),
                    pltpu.async_copy(tcl_hbm.at[b, rows, :], tcv, sem),
                    pltpu.async_copy(w_hbm.at[b, rows, :], wv, sem),
                    pltpu.async_copy(trm_hbm.at[b, rows, :], trmv, sem))

        def _drain(slot):
            for d in _issue_descs[slot]:
                d.wait()

        def _process(slot, accs):
            l0, l1, l2, l3, trv, tcv, wv, trmv, _ = slots[slot]

            @plsc.parallel_loop(0, CPIX // 16, 1, unroll=4, carry=accs)
            def _vec(j, acc):
                r = j >> 5
                cds = pl.ds((j & 31) * 16, 16)
                v0 = l0[r, cds]
                v1 = l1[r, cds]
                v2 = l2[r, cds]
                v3 = l3[r, cds]
                trl = trv[r, cds] > 0
                tcll = tcv[r, cds] > 0
                wgt = wv[r, cds]
                trml = trmv[r, cds] > 0
                d01 = v0 - v1
                ce_tr = _softplus_sc(jnp.where(trl, d01, -d01))
                d23 = v2 - v3
                ce_tcl = _softplus_sc(jnp.where(tcll, d23, -d23))
                pos = trl & trml
                neg = (~trl) & trml
                u = lax.bitcast_convert_type(ce_tr, jnp.int32)
                bk = jnp.clip(lax.shift_right_logical(u, 17) - BUCK_BASE,
                              0, NBUCK - 1)
                slot = (bk << 4) + lane
                plsc.addupdate_scatter(cnt_v, [slot], ones16, mask=neg)
                plsc.addupdate_scatter(sum_v, [slot], ce_tr, mask=neg)
                a0, a1, a2, a3, a4, a5 = acc
                return (a0 + jnp.where(pos, 1.0, 0.0),
                        a1 + jnp.where(pos, ce_tr, 0.0),
                        a2 + jnp.where(trl, 1.0, 0.0),
                        a3 + jnp.where(trl, ce_tr, 0.0),
                        a4 + jnp.where(neg, 1.0, 0.0),
                        a5 + jnp.where(pos, wgt * ce_tcl, 0.0))

            return _vec

        accs = (zeros16,) * 6
        _issue_descs = [_issue(0, 0), _issue(1, 1)]
        for k in range(NCHUNK):
            slot = k % 2
            _drain(slot)
            accs = _process(slot, accs)
            if k + 2 < NCHUNK:
                _issue_descs[slot] = _issue(slot, k + 2)
        for q in range(6):
            scal_v[q, :] = accs[q]
        pltpu.sync_copy(cnt_v, cnt_hbm.at[wid])
        pltpu.sync_copy(sum_v, sum_hbm.at[wid])
        pltpu.sync_copy(scal_v, scal_hbm.at[wid])

    return sc_kernel(inp, tr, tcl, w, trm)


def _final_body(cnt_ref, sum_ref, scal_ref, acc_ref, out_ref):
    cnt_b = jnp.sum(cnt_ref[...], axis=0, keepdims=True)
    sum_b = jnp.sum(sum_ref[...], axis=0, keepdims=True)
    bidx = lax.broadcasted_iota(jnp.int32, (1, NSLOT), 1) >> 4

    qidx = lax.broadcasted_iota(jnp.int32, (NW, 6, 16), 1)
    sc = scal_ref[...]

    def sq(q):
        return jnp.sum(jnp.where(qidx == q, sc, 0.0))

    s0, s1, s2, s3, s4, s5 = sq(0), sq(1), sq(2), sq(3), sq(4), sq(5)

    def arow(i):
        return jnp.sum(acc_ref[i:i + 1, :])

    s6, s7, s8, s9 = arow(0), arow(1), arow(2), arow(3)
    use_a = s0 > 0.0
    n_pos = jnp.where(use_a, s0, s2)
    loss_pos = jnp.where(use_a, s1, s3)
    kf = jnp.minimum(s4, jnp.floor(3.0 * n_pos))

    def bs_body(i, carry):
        lo, hi = carry
        mid = (lo + hi) // 2
        c = jnp.sum(jnp.where(bidx >= mid, cnt_b, 0.0))
        ge = c >= kf
        return (jnp.where(ge, mid, lo), jnp.where(ge, hi, mid))

    lo, _ = lax.fori_loop(0, 11, bs_body,
                          (jnp.int32(0), jnp.int32(NBUCK)))
    cnt_above = jnp.sum(jnp.where(bidx > lo, cnt_b, 0.0))
    sum_above = jnp.sum(jnp.where(bidx > lo, sum_b, 0.0))
    cb = jnp.sum(jnp.where(bidx == lo, cnt_b, 0.0))
    sb = jnp.sum(jnp.where(bidx == lo, sum_b, 0.0))
    r = kf - cnt_above
    part = jnp.where(r >= cb, sb, r * (sb / jnp.maximum(cb, 1.0)))
    tns = jnp.where(kf > 0.0, sum_above + part, 0.0)
    loss_tr = (loss_pos + tns) / (n_pos + kf)
    loss_tcl = s5 / s0
    nf = float(NPIX)
    vals = [loss_tr, loss_tcl, s6 / nf, s7 / nf, s8 / nf, s9 / nf]
    ri = lax.broadcasted_iota(jnp.int32, (8, 128), 0)
    li = lax.broadcasted_iota(jnp.int32, (8, 128), 1)
    o = jnp.zeros((8, 128), jnp.float32)
    for i, v in enumerate(vals):
        o += jnp.where((ri == 0) & (li == i), v, 0.0)
    out_ref[...] = o


def _stage3(cnt, summ, scal2, acc):
    return pl.pallas_call(
        _final_body,
        out_shape=jax.ShapeDtypeStruct((8, 128), jnp.float32),
    )(cnt, summ, scal2, acc)


def kernel(input, tr_mask, tcl_mask, tcl_weight, up_mask, down_mask,
           left_mask, right_mask, train_mask):
    cnt, summ, scal = _sc_main(input, tr_mask, tcl_mask, tcl_weight,
                               train_mask)
    acc = _stage1(input, up_mask, down_mask, left_mask, right_mask)
    out = _stage3(cnt, summ, scal, acc)
    return tuple(out[0, i] for i in range(6))


# speedup vs baseline: 1.0539x; 1.0539x over previous
import functools

import jax
import jax.numpy as jnp
from jax import lax
from jax.experimental import pallas as pl
from jax.experimental.pallas import tpu as pltpu
from jax.experimental.pallas import tpu_sc as plsc

BS, C, H, W = 8, 9, 512, 512
NPIX = BS * H * W

HB = 64
NHB = H // HB

MANT_BITS = 5
EXP_LO = 102
EXP_HI = 131
NBUCK = (EXP_HI - EXP_LO + 1) * (1 << MANT_BITS)
BUCK_BASE = EXP_LO << MANT_BITS
BIT_SHIFT = 23 - MANT_BITS

NW = 32
LANES = 16
NSLOT = NBUCK * LANES
QROWS = H // 4
CROWS = 8
CPIX = CROWS * W
NCHUNK = QROWS // CROWS


def _sl1(p, t):
    d = jnp.abs(p - t)
    return jnp.where(d < 1.0, 0.5 * d * d, d - 0.5)


def _main_body(c4_ref, c5_ref, c6_ref, c7_ref, up_ref, dn_ref, lf_ref, rt_ref,
               acc_ref):
    sums = [
        jnp.sum(_sl1(c4_ref[0, 0], up_ref[0])),
        jnp.sum(_sl1(c5_ref[0, 0], dn_ref[0])),
        jnp.sum(_sl1(c6_ref[0, 0], lf_ref[0])),
        jnp.sum(_sl1(c7_ref[0, 0], rt_ref[0])),
    ]
    rows = lax.broadcasted_iota(jnp.int32, (8, 128), 0)
    cols = lax.broadcasted_iota(jnp.int32, (8, 128), 1)
    p = jnp.zeros((8, 128), jnp.float32)
    for i, s in enumerate(sums):
        p += jnp.where((rows == i) & (cols == 0), s, 0.0)

    @pl.when((pl.program_id(0) == 0) & (pl.program_id(1) == 0))
    def _():
        acc_ref[...] = jnp.zeros_like(acc_ref)

    acc_ref[...] += p


def _stage1(inp, up, dn, lf, rt):
    def chan(c):
        return pl.BlockSpec((1, 1, HB, W), lambda b, h, c=c: (b, c, h, 0))

    msk = pl.BlockSpec((1, HB, W), lambda b, h: (b, h, 0))
    return pl.pallas_call(
        _main_body,
        grid=(BS, NHB),
        in_specs=[chan(4), chan(5), chan(6), chan(7)] + [msk] * 4,
        out_specs=pl.BlockSpec((8, 128), lambda b, h: (0, 0)),
        out_shape=jax.ShapeDtypeStruct((8, 128), jnp.float32),
    )(inp, inp, inp, inp, up, dn, lf, rt)


def _softplus_sc(m):
    e = jnp.exp(-jnp.abs(m))
    p = jnp.float32(-0.017414081461728154)
    for c in (0.08269125547625317, -0.19035436664917366, 0.3157473388836618,
              -0.49737322382928023, 0.9998476986074015, 1.4720198885558882e-06):
        p = p * e + jnp.float32(c)
    return jnp.maximum(m, 0.0) + p


def _sc_main(inp, tr, tcl, w, trm):
    mesh = plsc.VectorSubcoreMesh(core_axis_name="c", subcore_axis_name="s")
    fbuf = lambda: pltpu.VMEM((CROWS, W), jnp.float32)
    ibuf = lambda: pltpu.VMEM((CROWS, W), jnp.int32)

    @functools.partial(
        pl.kernel,
        out_type=(jax.ShapeDtypeStruct((NW, NSLOT), jnp.float32),
                  jax.ShapeDtypeStruct((NW, NSLOT), jnp.float32),
                  jax.ShapeDtypeStruct((NW, 6, 16), jnp.float32)),
        mesh=mesh,
        scratch_types=[
            fbuf(), fbuf(), fbuf(), fbuf(),
            fbuf(), fbuf(), fbuf(), fbuf(),
            ibuf(), ibuf(), fbuf(), ibuf(),
            ibuf(), ibuf(), fbuf(), ibuf(),
            pltpu.VMEM((NSLOT,), jnp.float32),
            pltpu.VMEM((NSLOT,), jnp.float32),
            pltpu.VMEM((6, 16), jnp.float32),
            pltpu.SemaphoreType.DMA,
            pltpu.SemaphoreType.DMA,
        ],
        compiler_params=pltpu.CompilerParams(needs_layout_passes=False),
    )
    def sc_kernel(in_hbm, tr_hbm, tcl_hbm, w_hbm, trm_hbm,
                  cnt_hbm, sum_hbm, scal_hbm,
                  l0a, l1a, l2a, l3a, l0b, l1b, l2b, l3b,
                  tra, tca, wa, trma, trb, tcb, wb, trmb,
                  cnt_v, sum_v, scal_v, sema, semb):
        wid = lax.axis_index("s") * 2 + lax.axis_index("c")
        b = wid // 4
        row0 = (wid % 4) * QROWS
        zeros16 = jnp.zeros((16,), jnp.float32)

        @plsc.parallel_loop(0, NSLOT // 16, 1, unroll=8)
        def _zero(i):
            cnt_v[pl.ds(i * 16, 16)] = zeros16
            sum_v[pl.ds(i * 16, 16)] = zeros16

        lane = lax.iota(jnp.int32, 16)
        ones16 = jnp.ones((16,), jnp.float32)
        slots = ((l0a, l1a, l2a, l3a, tra, tca, wa, trma, sema),
                 (l0b, l1b, l2b, l3b, trb, tcb, wb, trmb, semb))

        def _issue(slot, k):
            rows = pl.ds(row0 + k * CROWS, CROWS)
            l0, l1, l2, l3, trv, tcv, wv, trmv, sem = slots[slot]
            return (pltpu.async_copy(in_hbm.at[b, 0, rows, :], l0, sem),
                    pltpu.async_copy(in_hbm.at[b, 1, rows, :], l1, sem),
                    pltpu.async_copy(in_hbm.at[b, 2, rows, :], l2, sem),
                    pltpu.async_copy(in_hbm.at[b, 3, rows, :], l3, sem),
                    pltpu.async_copy(tr_hbm.at[b, rows, :], trv, sem),
                    pltpu.async_copy(tcl_hbm.at[b, rows, :], tcv, sem),
                    pltpu.async_copy(w_hbm.at[b, rows, :], wv, sem),
                    pltpu.async_copy(trm_hbm.at[b, rows, :], trmv, sem))

        def _drain(slot):
            for d in _issue_descs[slot]:
                d.wait()

        def _process(slot, accs):
            l0, l1, l2, l3, trv, tcv, wv, trmv, _ = slots[slot]

            @plsc.parallel_loop(0, CPIX // 16, 1, unroll=4, carry=accs)
            def _vec(j, acc):
                r = j >> 5
                cds = pl.ds((j & 31) * 16, 16)
                v0 = l0[r, cds]
                v1 = l1[r, cds]
                v2 = l2[r, cds]
                v3 = l3[r, cds]
                trl = trv[r, cds] > 0
                tcll = tcv[r, cds] > 0
                wgt = wv[r, cds]
                trml = trmv[r, cds] > 0
                d01 = v0 - v1
                ce_tr = _softplus_sc(jnp.where(trl, d01, -d01))
                d23 = v2 - v3
                ce_tcl = _softplus_sc(jnp.where(tcll, d23, -d23))
                pos = trl & trml
                neg = (~trl) & trml
                u = lax.bitcast_convert_type(ce_tr, jnp.int32)
                bk = jnp.clip(lax.shift_right_logical(u, BIT_SHIFT) - BUCK_BASE,
                              0, NBUCK - 1)
                slot = (bk << 4) + lane
                plsc.addupdate_scatter(cnt_v, [slot], ones16, mask=neg)
                plsc.addupdate_scatter(sum_v, [slot], ce_tr, mask=neg)
                a0, a1, a2, a3, a4, a5 = acc
                return (a0 + jnp.where(pos, 1.0, 0.0),
                        a1 + jnp.where(pos, ce_tr, 0.0),
                        a2 + jnp.where(trl, 1.0, 0.0),
                        a3 + jnp.where(trl, ce_tr, 0.0),
                        a4 + jnp.where(neg, 1.0, 0.0),
                        a5 + jnp.where(pos, wgt * ce_tcl, 0.0))

            return _vec

        accs = (zeros16,) * 6
        _issue_descs = [_issue(0, 0), _issue(1, 1)]
        for k in range(NCHUNK):
            slot = k % 2
            _drain(slot)
            accs = _process(slot, accs)
            if k + 2 < NCHUNK:
                _issue_descs[slot] = _issue(slot, k + 2)
        for q in range(6):
            scal_v[q, :] = accs[q]
        pltpu.sync_copy(cnt_v, cnt_hbm.at[wid])
        pltpu.sync_copy(sum_v, sum_hbm.at[wid])
        pltpu.sync_copy(scal_v, scal_hbm.at[wid])

    return sc_kernel(inp, tr, tcl, w, trm)


def _final_body(cnt_ref, sum_ref, scal_ref, acc_ref, out_ref):
    cnt_b = jnp.sum(cnt_ref[...], axis=0, keepdims=True)
    sum_b = jnp.sum(sum_ref[...], axis=0, keepdims=True)
    bidx = lax.broadcasted_iota(jnp.int32, (1, NSLOT), 1) >> 4

    qidx = lax.broadcasted_iota(jnp.int32, (NW, 6, 16), 1)
    sc = scal_ref[...]

    def sq(q):
        return jnp.sum(jnp.where(qidx == q, sc, 0.0))

    s0, s1, s2, s3, s4, s5 = sq(0), sq(1), sq(2), sq(3), sq(4), sq(5)

    def arow(i):
        return jnp.sum(acc_ref[i:i + 1, :])

    s6, s7, s8, s9 = arow(0), arow(1), arow(2), arow(3)
    use_a = s0 > 0.0
    n_pos = jnp.where(use_a, s0, s2)
    loss_pos = jnp.where(use_a, s1, s3)
    kf = jnp.minimum(s4, jnp.floor(3.0 * n_pos))

    def bs_body(i, carry):
        lo, hi = carry
        mid = (lo + hi) // 2
        c = jnp.sum(jnp.where(bidx >= mid, cnt_b, 0.0))
        ge = c >= kf
        return (jnp.where(ge, mid, lo), jnp.where(ge, hi, mid))

    lo, _ = lax.fori_loop(0, 11, bs_body,
                          (jnp.int32(0), jnp.int32(NBUCK)))
    cnt_above = jnp.sum(jnp.where(bidx > lo, cnt_b, 0.0))
    sum_above = jnp.sum(jnp.where(bidx > lo, sum_b, 0.0))
    cb = jnp.sum(jnp.where(bidx == lo, cnt_b, 0.0))
    sb = jnp.sum(jnp.where(bidx == lo, sum_b, 0.0))
    r = kf - cnt_above
    part = jnp.where(r >= cb, sb, r * (sb / jnp.maximum(cb, 1.0)))
    tns = jnp.where(kf > 0.0, sum_above + part, 0.0)
    loss_tr = (loss_pos + tns) / (n_pos + kf)
    loss_tcl = s5 / s0
    nf = float(NPIX)
    vals = [loss_tr, loss_tcl, s6 / nf, s7 / nf, s8 / nf, s9 / nf]
    ri = lax.broadcasted_iota(jnp.int32, (8, 128), 0)
    li = lax.broadcasted_iota(jnp.int32, (8, 128), 1)
    o = jnp.zeros((8, 128), jnp.float32)
    for i, v in enumerate(vals):
        o += jnp.where((ri == 0) & (li == i), v, 0.0)
    out_ref[...] = o


def _stage3(cnt, summ, scal2, acc):
    return pl.pallas_call(
        _final_body,
        out_shape=jax.ShapeDtypeStruct((8, 128), jnp.float32),
    )(cnt, summ, scal2, acc)


def kernel(input, tr_mask, tcl_mask, tcl_weight, up_mask, down_mask,
           left_mask, right_mask, train_mask):
    cnt, summ, scal = _sc_main(input, tr_mask, tcl_mask, tcl_weight,
                               train_mask)
    acc = _stage1(input, up_mask, down_mask, left_mask, right_mask)
    out = _stage3(cnt, summ, scal, acc)
    return tuple(out[0, i] for i in range(6))
